# Initial kernel scaffold; baseline (speedup 1.0000x reference)
import jax
import jax.numpy as jnp
from jax.experimental import pallas as pl


def kernel(x, edge_index, W1, b1, W2, b2):
    raise NotImplementedError("write your pallas kernel here")


# speedup vs baseline: 19.4998x; 19.4998x over previous
import functools

import jax
import jax.numpy as jnp
from jax import lax
from jax.experimental import pallas as pl
from jax.experimental.pallas import tpu as pltpu
from jax.experimental.pallas import tpu_sc as plsc

N_NODES = 10000
N_EDGES = 320000
D_IN = 128
D_HID = 128
D_OUT = 16

NC = 2
NS = 16
NW = NC * NS
CHUNK = 128
CPT = 79
EPT = CPT * CHUNK
E_PAD = NW * EPT
V_PAD = 10240
RPT = V_PAD // NS
RCH = RPT // CHUNK

_mesh = functools.partial(
    plsc.VectorSubcoreMesh,
    core_axis_name="c", subcore_axis_name="s", num_cores=NC, num_subcores=NS,
)


@functools.lru_cache(maxsize=None)
def _sc_edge_scatter(width):
    @functools.partial(
        pl.kernel,
        out_type=jax.ShapeDtypeStruct((NC, V_PAD, width), jnp.float32),
        mesh=_mesh(),
        compiler_params=pltpu.CompilerParams(use_tc_tiling_on_sc=False),
        scratch_types=[
            pltpu.VMEM((CPT, CHUNK), jnp.int32),
            pltpu.VMEM((CPT, CHUNK), jnp.int32),
            pltpu.VMEM((CHUNK, width), jnp.float32),
            pltpu.VMEM_SHARED((V_PAD, width), jnp.float32),
            pltpu.SemaphoreType.DMA,
        ],
    )
    def body(table_hbm, src_hbm, dst_hbm, zeros_hbm, out_hbm,
             src_v, dst_v, rows_v, agg_sh, sem):
        c = lax.axis_index("c")
        s = lax.axis_index("s")
        w = c * NS + s
        base = s * RPT

        pltpu.sync_copy(zeros_hbm, rows_v)
        def zero_body(r, carry):
            pltpu.sync_copy(rows_v, agg_sh.at[pl.ds(base + r * CHUNK, CHUNK), :])
            return carry
        lax.fori_loop(0, RCH, zero_body, 0)

        pltpu.sync_copy(src_hbm.at[w], src_v)
        pltpu.sync_copy(dst_hbm.at[w], dst_v)
        plsc.subcore_barrier()

        def edge_body(j, carry):
            pltpu.async_copy(table_hbm.at[src_v.at[j]], rows_v, sem).wait()
            pltpu.sync_copy(rows_v, agg_sh.at[dst_v.at[j]], add=True)
            return carry
        lax.fori_loop(0, CPT, edge_body, 0)
        plsc.subcore_barrier()

        def out_body(r, carry):
            off = base + r * CHUNK
            pltpu.sync_copy(agg_sh.at[pl.ds(off, CHUNK), :], rows_v)
            pltpu.sync_copy(rows_v, out_hbm.at[c, pl.ds(off, CHUNK), :])
            return carry
        lax.fori_loop(0, RCH, out_body, 0)

    return body


DEG_W = 16


@functools.lru_cache(maxsize=None)
def _sc_degree():
    @functools.partial(
        pl.kernel,
        out_type=jax.ShapeDtypeStruct((NC, V_PAD, DEG_W), jnp.float32),
        mesh=_mesh(),
        compiler_params=pltpu.CompilerParams(use_tc_tiling_on_sc=False),
        scratch_types=[
            pltpu.VMEM((CPT, CHUNK), jnp.int32),
            pltpu.VMEM((CHUNK, DEG_W), jnp.float32),
            pltpu.VMEM((CHUNK, DEG_W), jnp.float32),
            pltpu.VMEM_SHARED((V_PAD, DEG_W), jnp.float32),
        ],
    )
    def body(dst_hbm, zeros_hbm, ones_hbm, out_hbm, dst_v, buf_v, ones_v,
             deg_sh):
        c = lax.axis_index("c")
        s = lax.axis_index("s")
        w = c * NS + s
        base = s * RPT

        pltpu.sync_copy(zeros_hbm, buf_v)
        def zero_body(r, carry):
            pltpu.sync_copy(buf_v, deg_sh.at[pl.ds(base + r * CHUNK, CHUNK), :])
            return carry
        lax.fori_loop(0, RCH, zero_body, 0)

        pltpu.sync_copy(ones_hbm, ones_v)
        pltpu.sync_copy(dst_hbm.at[w], dst_v)
        plsc.subcore_barrier()

        def edge_body(j, carry):
            pltpu.sync_copy(ones_v, deg_sh.at[dst_v.at[j]], add=True)
            return carry
        lax.fori_loop(0, CPT, edge_body, 0)
        plsc.subcore_barrier()

        def out_body(r, carry):
            off = base + r * CHUNK
            pltpu.sync_copy(deg_sh.at[pl.ds(off, CHUNK), :], buf_v)
            pltpu.sync_copy(buf_v, out_hbm.at[c, pl.ds(off, CHUNK), :])
            return carry
        lax.fori_loop(0, RCH, out_body, 0)

    return body


def _tc_layer1(x, W1, deg_parts):
    B = 1000

    def body(x_ref, w_ref, dp_ref, hn_ref, dis_ref):
        h = jnp.dot(x_ref[...], w_ref[...], preferred_element_type=jnp.float32)
        deg = dp_ref[0][:, :1] + dp_ref[1][:, :1] + 1.0
        dis = lax.rsqrt(deg)
        hn_ref[...] = h * dis
        dis_ref[...] = jnp.broadcast_to(dis, (B, DEG_W))

    return pl.pallas_call(
        body,
        grid=(N_NODES // B,),
        in_specs=[
            pl.BlockSpec((B, D_IN), lambda i: (i, 0)),
            pl.BlockSpec((D_IN, D_HID), lambda i: (0, 0)),
            pl.BlockSpec((NC, B, DEG_W), lambda i: (0, i, 0)),
        ],
        out_specs=[
            pl.BlockSpec((B, D_HID), lambda i: (i, 0)),
            pl.BlockSpec((B, DEG_W), lambda i: (i, 0)),
        ],
        out_shape=[
            jax.ShapeDtypeStruct((N_NODES, D_HID), jnp.float32),
            jax.ShapeDtypeStruct((N_NODES, DEG_W), jnp.float32),
        ],
    )(x, W1, deg_parts)


def _tc_layer2(agg_parts, hn1, dis8, b1, W2):
    B = 1000

    def body(ap_ref, hn_ref, dis_ref, b1_ref, w2_ref, out_ref):
        s = ap_ref[0] + ap_ref[1] + hn_ref[...]
        dis = dis_ref[:, :1]
        y = jnp.maximum(s * dis + b1_ref[...], 0.0)
        out_ref[...] = jnp.dot(y, w2_ref[...],
                               preferred_element_type=jnp.float32) * dis

    return pl.pallas_call(
        body,
        grid=(N_NODES // B,),
        in_specs=[
            pl.BlockSpec((NC, B, D_HID), lambda i: (0, i, 0)),
            pl.BlockSpec((B, D_HID), lambda i: (i, 0)),
            pl.BlockSpec((B, DEG_W), lambda i: (i, 0)),
            pl.BlockSpec((1, D_HID), lambda i: (0, 0)),
            pl.BlockSpec((D_HID, D_OUT), lambda i: (0, 0)),
        ],
        out_specs=pl.BlockSpec((B, D_OUT), lambda i: (i, 0)),
        out_shape=jax.ShapeDtypeStruct((N_NODES, D_OUT), jnp.float32),
    )(agg_parts, hn1, dis8, b1, W2)


def _tc_final(agg_parts, hn2, dis8, b2):
    B = 1000

    def body(ap_ref, hn_ref, dis_ref, b2_ref, out_ref):
        z = (ap_ref[0] + ap_ref[1] + hn_ref[...]) * dis_ref[:, :1] + b2_ref[...]
        zm = z - jnp.max(z, axis=1, keepdims=True)
        out_ref[...] = zm - jnp.log(
            jnp.sum(jnp.exp(zm), axis=1, keepdims=True))

    return pl.pallas_call(
        body,
        grid=(N_NODES // B,),
        in_specs=[
            pl.BlockSpec((NC, B, D_OUT), lambda i: (0, i, 0)),
            pl.BlockSpec((B, D_OUT), lambda i: (i, 0)),
            pl.BlockSpec((B, DEG_W), lambda i: (i, 0)),
            pl.BlockSpec((1, D_OUT), lambda i: (0, 0)),
        ],
        out_specs=pl.BlockSpec((B, D_OUT), lambda i: (i, 0)),
        out_shape=jax.ShapeDtypeStruct((N_NODES, D_OUT), jnp.float32),
    )(agg_parts, hn2, dis8, b2)


def kernel(x, edge_index, W1, b1, W2, b2):
    ei = edge_index.astype(jnp.int32)
    pad = E_PAD - N_EDGES
    src3 = jnp.concatenate(
        [ei[0], jnp.zeros((pad,), jnp.int32)]).reshape(NW, CPT, CHUNK)
    dst3 = jnp.concatenate(
        [ei[1], jnp.full((pad,), N_NODES, jnp.int32)]).reshape(NW, CPT, CHUNK)

    zeros_w = jnp.zeros((CHUNK, DEG_W), jnp.float32)
    ones_w = jnp.ones((CHUNK, DEG_W), jnp.float32)
    deg_parts = _sc_degree()(dst3, zeros_w, ones_w)

    hn1, dis8 = _tc_layer1(x, W1, deg_parts)
    agg1 = _sc_edge_scatter(D_HID)(
        hn1, src3, dst3, jnp.zeros((CHUNK, D_HID), jnp.float32))

    hn2 = _tc_layer2(agg1, hn1, dis8, b1.reshape(1, D_HID), W2)
    agg2 = _sc_edge_scatter(D_OUT)(
        hn2, src3, dst3, jnp.zeros((CHUNK, D_OUT), jnp.float32))

    return _tc_final(agg2, hn2, dis8, b2.reshape(1, D_OUT))
